# baseline (speedup 1.0000x reference)
import functools

import jax
import jax.numpy as jnp
from jax import lax
from jax.experimental import pallas as pl
from jax.experimental.pallas import tpu as pltpu
from jax.experimental.pallas import tpu_sc as plsc

NC = 2
NS = 16
L = 16
NW = NC * NS
W = 128
UNROLL = 8
NACC = 4


@functools.lru_cache(maxsize=4)
def _build_sc(B, D, NNEG, C):
    b_per_w = B // NW
    n_chunks = b_per_w // C
    n_idx = NNEG + 2
    n_groups = C // L
    mesh = plsc.VectorSubcoreMesh(core_axis_name="c", subcore_axis_name="s",
                                  num_cores=NC, num_subcores=NS)

    @functools.partial(
        pl.kernel,
        out_type=(
            jax.ShapeDtypeStruct((B,), jnp.float32),
            jax.ShapeDtypeStruct((B,), jnp.float32),
        ),
        mesh=mesh,
        compiler_params=pltpu.CompilerParams(needs_layout_passes=False),
        scratch_types=[
            pltpu.VMEM((n_idx, b_per_w), jnp.int32),
            pltpu.VMEM((n_idx, b_per_w), jnp.int32),
            pltpu.VMEM((C, W), jnp.float32),
            pltpu.VMEM((C, W), jnp.float32),
            pltpu.VMEM((C, W), jnp.float32),
            pltpu.VMEM((C, W), jnp.float32),
            pltpu.VMEM((b_per_w,), jnp.float32),
            pltpu.VMEM((b_per_w,), jnp.float32),
            pltpu.SemaphoreType.DMA,
            pltpu.SemaphoreType.DMA,
            pltpu.SemaphoreType.DMA,
        ],
    )
    def sc_kernel(u2_hbm, v2_hbm, idx_hbm, score_hbm, nscore_hbm,
                  islab, rslab, urows, vrows, nbufa, nbufb,
                  sbuf, nsbuf, semuv, sema, semb):
        wid = lax.axis_index("s") * NC + lax.axis_index("c")
        base = wid * b_per_w
        lane = lax.iota(jnp.int32, L)

        pltpu.sync_copy(idx_hbm.at[:, pl.ds(base, b_per_w)], islab)

        def shift_body(i, _):
            for r in range(n_idx):
                v = islab[r, pl.ds(i * L, L)]
                rslab[r, pl.ds(i * L, L)] = v >> 1
            return 0

        lax.fori_loop(0, b_per_w // L, shift_body, 0)

        nbufs = (nbufa, nbufb)
        sems = (sema, semb)

        def fire_neg(n, cbase):
            return pltpu.async_copy(
                v2_hbm.at[rslab.at[2 + n, pl.ds(cbase, C)]],
                nbufs[n % 2], sems[n % 2])

        def chunk_body(ch, _):
            cbase = ch * C
            cp_uv = [
                pltpu.async_copy(
                    u2_hbm.at[rslab.at[0, pl.ds(cbase, C)]], urows, semuv),
                pltpu.async_copy(
                    v2_hbm.at[rslab.at[1, pl.ds(cbase, C)]], vrows, semuv),
            ]
            cp = fire_neg(0, cbase)

            for c in cp_uv:
                c.wait()

            def dot_g_body(g, _):
                gl = g * L
                gb = cbase + gl
                bvec = gl + lane
                hu = (islab[0, pl.ds(gb, L)] & 1) << 6
                hv = (islab[1, pl.ds(gb, L)] & 1) << 6

                def d_body(di, accs):
                    dd = di * UNROLL
                    accs = list(accs)
                    for k in range(UNROLL):
                        uvec = plsc.load_gather(urows, [bvec, hu + dd + k])
                        vvec = plsc.load_gather(vrows, [bvec, hv + dd + k])
                        accs[k % NACC] = accs[k % NACC] + uvec * vvec
                    return tuple(accs)

                zero = jnp.zeros((L,), jnp.float32)
                accs = lax.fori_loop(0, D // UNROLL, d_body, (zero,) * NACC)
                sbuf[pl.ds(gb, L)] = sum(accs)
                return 0

            lax.fori_loop(0, n_groups, dot_g_body, 0)

            for n in range(NNEG):
                cp.wait()
                if n + 1 < NNEG:
                    cp = fire_neg(n + 1, cbase)
                nbuf = nbufs[n % 2]

                def g_body(g, _, nbuf=nbuf, n=n):
                    gl = g * L
                    gb = cbase + gl
                    bvec = gl + lane
                    hu = (islab[0, pl.ds(gb, L)] & 1) << 6
                    hn = (islab[2 + n, pl.ds(gb, L)] & 1) << 6

                    def acc_body(di, accs):
                        dd = di * UNROLL
                        accs = list(accs)
                        for k in range(UNROLL):
                            uvec = plsc.load_gather(
                                urows, [bvec, hu + dd + k])
                            nvec = plsc.load_gather(nbuf, [bvec, hn + dd + k])
                            accs[k % NACC] = accs[k % NACC] + uvec * nvec
                        return tuple(accs)

                    zero = jnp.zeros((L,), jnp.float32)
                    accs = lax.fori_loop(0, D // UNROLL, acc_body,
                                         (zero,) * NACC)
                    ns = sum(accs)
                    if n == 0:
                        nsbuf[pl.ds(gb, L)] = ns
                    else:
                        nsbuf[pl.ds(gb, L)] += ns
                    return 0

                lax.fori_loop(0, n_groups, g_body, 0)
            return 0

        lax.fori_loop(0, n_chunks, chunk_body, 0)
        pltpu.sync_copy(sbuf, score_hbm.at[pl.ds(base, b_per_w)])
        pltpu.sync_copy(nsbuf, nscore_hbm.at[pl.ds(base, b_per_w)])

    return sc_kernel


def _tc_reduce_body(s_ref, t_ref, out_ref):
    s = s_ref[...]
    t = -t_ref[...]
    ls = jnp.minimum(s, 0.0) - jnp.log1p(jnp.exp(-jnp.abs(s)))
    lt = jnp.minimum(t, 0.0) - jnp.log1p(jnp.exp(-jnp.abs(t)))
    out_ref[0, 0] = jnp.sum(ls) + jnp.sum(lt)


@functools.lru_cache(maxsize=4)
def _build_tc(B):
    rows = B // 128
    return pl.pallas_call(
        _tc_reduce_body,
        out_shape=jax.ShapeDtypeStruct((1, 1), jnp.float32),
        in_specs=[pl.BlockSpec((rows, 128), lambda: (0, 0)),
                  pl.BlockSpec((rows, 128), lambda: (0, 0))],
        out_specs=pl.BlockSpec(memory_space=pltpu.SMEM),
    )


def kernel(u_pos, v_pos, v_neg, batch_size, U, V):
    B, = u_pos.shape
    _, NNEG = v_neg.shape
    VOCAB, D = U.shape
    U2 = U.reshape(VOCAB * D // W, W)
    V2 = V.reshape(VOCAB * D // W, W)
    idxall = jnp.concatenate(
        [u_pos[None].astype(jnp.int32), v_pos[None].astype(jnp.int32),
         v_neg.astype(jnp.int32).T], axis=0)

    score, neg_score = _build_sc(B, D, NNEG, 128)(U2, V2, idxall)
    total = _build_tc(B)(score.reshape(B // 128, 128),
                         neg_score.reshape(B // 128, 128))
    return -total[0, 0] / batch_size


# speedup vs baseline: 1.4585x; 1.4585x over previous
import functools

import jax
import jax.numpy as jnp
from jax import lax
from jax.experimental import pallas as pl
from jax.experimental.pallas import tpu as pltpu
from jax.experimental.pallas import tpu_sc as plsc

NC = 2
NS = 16
L = 16
NW = NC * NS
W = 128
UNROLL = 8


@functools.lru_cache(maxsize=4)
def _build_sc(B, D, NNEG, C):
    b_per_w = B // NW
    n_chunks = b_per_w // C
    n_idx = NNEG + 2
    n_groups = C // L
    mesh = plsc.VectorSubcoreMesh(core_axis_name="c", subcore_axis_name="s",
                                  num_cores=NC, num_subcores=NS)

    @functools.partial(
        pl.kernel,
        out_type=(
            jax.ShapeDtypeStruct((B,), jnp.float32),
            jax.ShapeDtypeStruct((B,), jnp.float32),
        ),
        mesh=mesh,
        compiler_params=pltpu.CompilerParams(needs_layout_passes=False),
        scratch_types=[
            pltpu.VMEM((n_idx, b_per_w), jnp.int32),
            pltpu.VMEM((n_idx, b_per_w), jnp.int32),
            pltpu.VMEM((C, W), jnp.float32),
            pltpu.VMEM((C, W), jnp.float32),
            pltpu.VMEM((C, W), jnp.float32),
            pltpu.VMEM((C, W), jnp.float32),
            pltpu.VMEM((D, C), jnp.float32),
            pltpu.VMEM((b_per_w,), jnp.float32),
            pltpu.VMEM((b_per_w,), jnp.float32),
            pltpu.SemaphoreType.DMA,
            pltpu.SemaphoreType.DMA,
            pltpu.SemaphoreType.DMA,
        ],
    )
    def sc_kernel(u2_hbm, v2_hbm, idx_hbm, score_hbm, nscore_hbm,
                  islab, rslab, urows, vrows, nbufa, nbufb, nsum,
                  sbuf, nsbuf, semuv, sema, semb):
        wid = lax.axis_index("s") * NC + lax.axis_index("c")
        base = wid * b_per_w
        lane = lax.iota(jnp.int32, L)

        pltpu.sync_copy(idx_hbm.at[:, pl.ds(base, b_per_w)], islab)

        def shift_body(i, _):
            for r in range(n_idx):
                v = islab[r, pl.ds(i * L, L)]
                rslab[r, pl.ds(i * L, L)] = v >> 1
            return 0

        lax.fori_loop(0, b_per_w // L, shift_body, 0)

        nbufs = (nbufa, nbufb)
        sems = (sema, semb)

        def fire_neg(n, cbase):
            return pltpu.async_copy(
                v2_hbm.at[rslab.at[2 + n, pl.ds(cbase, C)]],
                nbufs[n % 2], sems[n % 2])

        def chunk_body(ch, _):
            cbase = ch * C
            cp_uv = [
                pltpu.async_copy(
                    u2_hbm.at[rslab.at[0, pl.ds(cbase, C)]], urows, semuv),
                pltpu.async_copy(
                    v2_hbm.at[rslab.at[1, pl.ds(cbase, C)]], vrows, semuv),
            ]
            cp = fire_neg(0, cbase)

            for n in range(NNEG):
                cp.wait()
                if n + 1 < NNEG:
                    cp = fire_neg(n + 1, cbase)
                nbuf = nbufs[n % 2]

                def g_body(g, _, nbuf=nbuf, n=n):
                    gl = g * L
                    bvec = gl + lane
                    hn = (islab[2 + n, pl.ds(cbase + gl, L)] & 1) << 6

                    def acc_body(di, _):
                        dd = di * UNROLL
                        rots = [((dd + k) + lane) & (D - 1)
                                for k in range(UNROLL)]
                        vs = [plsc.load_gather(nbuf, [bvec, hn + rots[k]])
                              for k in range(UNROLL)]
                        for k in range(UNROLL):
                            if n == 0:
                                nsum[dd + k, pl.ds(gl, L)] = vs[k]
                            else:
                                nsum[dd + k, pl.ds(gl, L)] += vs[k]
                        return 0

                    lax.fori_loop(0, D // UNROLL, acc_body, 0)
                    return 0

                lax.fori_loop(0, n_groups, g_body, 0)

            for c in cp_uv:
                c.wait()

            def dot_g_body(g, _):
                gl = g * L
                gb = cbase + gl
                bvec = gl + lane
                hu = (islab[0, pl.ds(gb, L)] & 1) << 6
                hv = (islab[1, pl.ds(gb, L)] & 1) << 6

                def d_body(di, carry):
                    sc, ns = carry
                    dd = di * UNROLL
                    for k in range(UNROLL):
                        rot = ((dd + k) + lane) & (D - 1)
                        uvec = plsc.load_gather(urows, [bvec, hu + rot])
                        vvec = plsc.load_gather(vrows, [bvec, hv + rot])
                        nsvec = nsum[dd + k, pl.ds(gl, L)]
                        sc = sc + uvec * vvec
                        ns = ns + uvec * nsvec
                    return sc, ns

                zero = jnp.zeros((L,), jnp.float32)
                score, nscore = lax.fori_loop(0, D // UNROLL, d_body,
                                              (zero, zero))
                sbuf[pl.ds(gb, L)] = score
                nsbuf[pl.ds(gb, L)] = nscore
                return 0

            lax.fori_loop(0, n_groups, dot_g_body, 0)
            return 0

        lax.fori_loop(0, n_chunks, chunk_body, 0)
        pltpu.sync_copy(sbuf, score_hbm.at[pl.ds(base, b_per_w)])
        pltpu.sync_copy(nsbuf, nscore_hbm.at[pl.ds(base, b_per_w)])

    return sc_kernel


def _tc_reduce_body(s_ref, t_ref, out_ref):
    s = s_ref[...]
    t = -t_ref[...]
    ls = jnp.minimum(s, 0.0) - jnp.log1p(jnp.exp(-jnp.abs(s)))
    lt = jnp.minimum(t, 0.0) - jnp.log1p(jnp.exp(-jnp.abs(t)))
    out_ref[0, 0] = jnp.sum(ls) + jnp.sum(lt)


@functools.lru_cache(maxsize=4)
def _build_tc(B):
    rows = B // 128
    return pl.pallas_call(
        _tc_reduce_body,
        out_shape=jax.ShapeDtypeStruct((1, 1), jnp.float32),
        in_specs=[pl.BlockSpec((rows, 128), lambda: (0, 0)),
                  pl.BlockSpec((rows, 128), lambda: (0, 0))],
        out_specs=pl.BlockSpec(memory_space=pltpu.SMEM),
    )


def kernel(u_pos, v_pos, v_neg, batch_size, U, V):
    B, = u_pos.shape
    _, NNEG = v_neg.shape
    VOCAB, D = U.shape
    U2 = U.reshape(VOCAB * D // W, W)
    V2 = V.reshape(VOCAB * D // W, W)
    idxall = jnp.concatenate(
        [u_pos[None].astype(jnp.int32), v_pos[None].astype(jnp.int32),
         v_neg.astype(jnp.int32).T], axis=0)

    score, neg_score = _build_sc(B, D, NNEG, 128)(U2, V2, idxall)
    total = _build_tc(B)(score.reshape(B // 128, 128),
                         neg_score.reshape(B // 128, 128))
    return -total[0, 0] / batch_size
